# baseline (speedup 1.0000x reference)
import jax
import jax.numpy as jnp
from jax import lax
from jax.experimental import pallas as pl
from jax.experimental.pallas import tpu as pltpu
from jax.experimental.pallas import tpu_sc as plsc

_VOCAB = 100000
_EMB = 64
_B = 4096
_L = 200

_N = _B * _L
_NC = 2
_NS = 16
_NW = _NC * _NS
_BPW = _B // _NW
_LP = _L + 1


def kernel(table, input_tensor):
    flat_idx = input_tensor.reshape(_N)
    mesh = plsc.VectorSubcoreMesh(core_axis_name="c", subcore_axis_name="s")

    @pl.kernel(
        out_type=jax.ShapeDtypeStruct((_B, _EMB, _L), jnp.float32),
        mesh=mesh,
        compiler_params=pltpu.CompilerParams(use_tc_tiling_on_sc=False,
                                             needs_layout_passes=False),
        scratch_types=[
            pltpu.VMEM((_L,), jnp.int32),
            pltpu.VMEM((_L, _EMB), jnp.float32),
            pltpu.VMEM((_EMB, _LP), jnp.float32),
            pltpu.SemaphoreType.DMA,
        ],
    )
    def sc_kernel(table_hbm, idx_hbm, out_hbm, idx_v, rows_v, outT_v, sem):
        wid = lax.axis_index("s") * _NC + lax.axis_index("c")
        b0 = wid * _BPW
        iota16 = lax.iota(jnp.int32, 16)
        row_ids = [iota16 + h0 * 16 for h0 in range(_EMB // 16)]

        @pl.loop(0, _BPW)
        def _(bi):
            b = b0 + bi
            pltpu.sync_copy(idx_hbm.at[pl.ds(b * _L, _L)], idx_v)
            pltpu.async_copy(table_hbm.at[idx_v], rows_v, sem).wait()

            @pl.loop(0, _L)
            def _(l):
                col = jnp.full((16,), l, jnp.int32)
                for h0 in range(_EMB // 16):
                    x = rows_v[l, pl.ds(h0 * 16, 16)]
                    plsc.store_scatter(outT_v, [row_ids[h0], col], x)

            pltpu.sync_copy(outT_v.at[:, pl.ds(0, _L)], out_hbm.at[b])

    return sc_kernel(table, flat_idx)


# speedup vs baseline: 1.8932x; 1.8932x over previous
import jax
import jax.numpy as jnp
from jax import lax
from jax.experimental import pallas as pl
from jax.experimental.pallas import tpu as pltpu
from jax.experimental.pallas import tpu_sc as plsc

_VOCAB = 100000
_EMB = 64
_B = 4096
_L = 200

_NC = 2
_NS = 16
_NW = _NC * _NS
_LT = _L // 8
_BT = _B // 128
_NBLK = _LT * 2 * _BT
_BPW = _NBLK // _NW
_CH = 512
_PITCH = 513


def kernel(table, input_tensor):
    idx_r = (
        input_tensor.reshape(_BT, 128, _LT, 2, 4)
        .transpose(2, 3, 0, 1, 4)
        .reshape(_NBLK * _CH)
    )
    mesh = plsc.VectorSubcoreMesh(core_axis_name="c", subcore_axis_name="s")

    @pl.kernel(
        out_type=jax.ShapeDtypeStruct((_EMB, _LT, _BT, 8, 128), jnp.float32),
        mesh=mesh,
        compiler_params=pltpu.CompilerParams(use_tc_tiling_on_sc=False,
                                             needs_layout_passes=False),
        scratch_types=[
            pltpu.VMEM((_CH,), jnp.int32),
            pltpu.VMEM((_CH, _EMB), jnp.float32),
            pltpu.VMEM((_EMB, _PITCH), jnp.float32),
            pltpu.SemaphoreType.DMA,
        ],
    )
    def sc_kernel(table_hbm, idx_hbm, out_hbm, idx_v, rows_v, outT_v, sem):
        wid = lax.axis_index("s") * _NC + lax.axis_index("c")
        iota16 = lax.iota(jnp.int32, 16)
        row_ids = [iota16 + h0 * 16 for h0 in range(_EMB // 16)]

        @pl.loop(0, _BPW)
        def _(gi):
            g = wid * _BPW + gi
            bt = g % _BT
            lh = g // _BT
            half = lh % 2
            lt = lh // 2

            pltpu.sync_copy(idx_hbm.at[pl.ds(g * _CH, _CH)], idx_v)
            pltpu.async_copy(table_hbm.at[idx_v], rows_v, sem).wait()

            @pl.loop(0, _CH)
            def _(r):
                col = jnp.full((16,), (r % 4) * 128 + (r // 4), jnp.int32)
                for h0 in range(_EMB // 16):
                    x = rows_v[r, pl.ds(h0 * 16, 16)]
                    plsc.store_scatter(outT_v, [row_ids[h0], col], x)

            for lp in range(4):
                pltpu.sync_copy(
                    outT_v.at[:, pl.ds(lp * 128, 128)],
                    out_hbm.at[:, lt, bt, half * 4 + lp, :],
                )

    out5 = sc_kernel(table, idx_r)
    return out5.transpose(2, 4, 0, 1, 3).reshape(_B, _EMB, _L)


# speedup vs baseline: 2.1065x; 1.1126x over previous
import jax
import jax.numpy as jnp
from jax import lax
from jax.experimental import pallas as pl
from jax.experimental.pallas import tpu as pltpu
from jax.experimental.pallas import tpu_sc as plsc

_VOCAB = 100000
_EMB = 64
_B = 4096
_L = 200

_NC = 2
_NS = 16
_NW = _NC * _NS
_LT = _L // 8
_BTS = _B // 128
_BG = _B // 512
_NBLK = _L * _BG
_BPW = _NBLK // _NW
_CH = 512
_PITCH = 521


def kernel(table, input_tensor):
    idx_t = input_tensor.T
    mesh = plsc.VectorSubcoreMesh(core_axis_name="c", subcore_axis_name="s")

    @pl.kernel(
        out_type=jax.ShapeDtypeStruct((_EMB, _LT, _BTS, 8, 128), jnp.float32),
        mesh=mesh,
        compiler_params=pltpu.CompilerParams(use_tc_tiling_on_sc=False,
                                             needs_layout_passes=False),
        scratch_types=[
            pltpu.VMEM((_CH,), jnp.int32),
            pltpu.VMEM((_CH,), jnp.int32),
            pltpu.VMEM((_CH, _EMB), jnp.float32),
            pltpu.VMEM((_CH, _EMB), jnp.float32),
            pltpu.VMEM((_EMB, _PITCH), jnp.float32),
            pltpu.SemaphoreType.DMA,
            pltpu.SemaphoreType.DMA,
        ],
    )
    def sc_kernel(table_hbm, idx_hbm, out_hbm,
                  idx0, idx1, rows0, rows1, outT_v, sem0, sem1):
        wid = lax.axis_index("s") * _NC + lax.axis_index("c")
        g0 = wid * _BPW
        iota16 = lax.iota(jnp.int32, 16)
        row_ids = [iota16 + h0 * 16 for h0 in range(_EMB // 16)]

        def issue(g, idx_v, rows_v, sem):
            l = g // _BG
            bg = g % _BG
            pltpu.sync_copy(idx_hbm.at[l, pl.ds(bg * _CH, _CH)], idx_v)
            pltpu.async_copy(table_hbm.at[idx_v], rows_v, sem)

        def finish(g, idx_v, rows_v, sem):
            l = g // _BG
            bg = g % _BG
            pltpu.make_async_copy(table_hbm.at[idx_v], rows_v, sem).wait()

            @pl.loop(0, _CH)
            def _(r):
                col = jnp.full((16,), r, jnp.int32)
                for h0 in range(_EMB // 16):
                    x = rows_v[r, pl.ds(h0 * 16, 16)]
                    plsc.store_scatter(outT_v, [row_ids[h0], col], x)

            for btp in range(4):
                pltpu.sync_copy(
                    outT_v.at[:, pl.ds(btp * 128, 128)],
                    out_hbm.at[:, l // 8, bg * 4 + btp, l % 8, :],
                )

        issue(g0 + 0, idx0, rows0, sem0)
        issue(g0 + 1, idx1, rows1, sem1)

        @pl.loop(0, _BPW // 2 - 1)
        def _(i):
            e = g0 + 2 * i
            finish(e, idx0, rows0, sem0)
            issue(e + 2, idx0, rows0, sem0)
            finish(e + 1, idx1, rows1, sem1)
            issue(e + 3, idx1, rows1, sem1)

        finish(g0 + _BPW - 2, idx0, rows0, sem0)
        finish(g0 + _BPW - 1, idx1, rows1, sem1)

    out5 = sc_kernel(table, idx_t)
    return out5.transpose(2, 4, 0, 1, 3).reshape(_B, _EMB, _L)


# speedup vs baseline: 5.1751x; 2.4568x over previous
import jax
import jax.numpy as jnp
from jax import lax
from jax.experimental import pallas as pl
from jax.experimental.pallas import tpu as pltpu
from jax.experimental.pallas import tpu_sc as plsc

_VOCAB = 100000
_EMB = 64
_B = 4096
_L = 200

_NC = 2
_NS = 16
_NW = _NC * _NS
_LT = _L // 8
_BTS = _B // 128
_BG = _B // 512
_NBLK = _L * _BG
_BPW = _NBLK // _NW
_CH = 512
_PITCH = 521


def kernel(table, input_tensor):
    idx_t = input_tensor.T
    mesh = plsc.VectorSubcoreMesh(core_axis_name="c", subcore_axis_name="s")

    @pl.kernel(
        out_type=jax.ShapeDtypeStruct((_EMB, _LT, _BTS, 8, 128), jnp.float32),
        mesh=mesh,
        compiler_params=pltpu.CompilerParams(use_tc_tiling_on_sc=False,
                                             needs_layout_passes=False),
        scratch_types=[
            pltpu.VMEM((_CH,), jnp.int32),
            pltpu.VMEM((_CH,), jnp.int32),
            pltpu.VMEM((_CH, _EMB), jnp.float32),
            pltpu.VMEM((_CH, _EMB), jnp.float32),
            pltpu.VMEM((_EMB, _PITCH), jnp.float32),
            pltpu.SemaphoreType.DMA,
            pltpu.SemaphoreType.DMA,
        ],
    )
    def sc_kernel(table_hbm, idx_hbm, out_hbm,
                  idx0, idx1, rows0, rows1, outT_v, sem0, sem1):
        wid = lax.axis_index("s") * _NC + lax.axis_index("c")
        g0 = wid * _BPW
        iota16 = lax.iota(jnp.int32, 16)
        row_ids = [iota16 + h0 * 16 for h0 in range(_EMB // 16)]

        def issue(g, idx_v, rows_v, sem):
            l = g // _BG
            bg = g % _BG
            pltpu.sync_copy(idx_hbm.at[l, pl.ds(bg * _CH, _CH)], idx_v)
            pltpu.async_copy(table_hbm.at[idx_v], rows_v, sem)

        def finish(g, idx_v, rows_v, sem):
            l = g // _BG
            bg = g % _BG
            pltpu.make_async_copy(table_hbm.at[idx_v], rows_v, sem).wait()

            @plsc.parallel_loop(0, _CH, unroll=8)
            def _(r):
                col = jnp.full((16,), r, jnp.int32)
                for h0 in range(_EMB // 16):
                    x = rows_v[r, pl.ds(h0 * 16, 16)]
                    plsc.store_scatter(outT_v, [row_ids[h0], col], x)

            for btp in range(4):
                pltpu.sync_copy(
                    outT_v.at[:, pl.ds(btp * 128, 128)],
                    out_hbm.at[:, l // 8, bg * 4 + btp, l % 8, :],
                )

        issue(g0 + 0, idx0, rows0, sem0)
        issue(g0 + 1, idx1, rows1, sem1)

        @pl.loop(0, _BPW // 2 - 1)
        def _(i):
            e = g0 + 2 * i
            finish(e, idx0, rows0, sem0)
            issue(e + 2, idx0, rows0, sem0)
            finish(e + 1, idx1, rows1, sem1)
            issue(e + 3, idx1, rows1, sem1)

        finish(g0 + _BPW - 2, idx0, rows0, sem0)
        finish(g0 + _BPW - 1, idx1, rows1, sem1)

    out5 = sc_kernel(table, idx_t)
    return out5.transpose(2, 4, 0, 1, 3).reshape(_B, _EMB, _L)


# speedup vs baseline: 5.2164x; 1.0080x over previous
import jax
import jax.numpy as jnp
from jax import lax
from jax.experimental import pallas as pl
from jax.experimental.pallas import tpu as pltpu
from jax.experimental.pallas import tpu_sc as plsc

_VOCAB = 100000
_EMB = 64
_B = 4096
_L = 200

_NC = 2
_NS = 16
_NW = _NC * _NS
_LT = _L // 8
_BTS = _B // 128
_BG = _B // 512
_NBLK = _L * _BG
_BPW = _NBLK // _NW
_CH = 512
_PITCH = 521


def kernel(table, input_tensor):
    idx_t = input_tensor.T
    mesh = plsc.VectorSubcoreMesh(core_axis_name="c", subcore_axis_name="s")

    @pl.kernel(
        out_type=jax.ShapeDtypeStruct((_EMB, _LT, _BTS, 8, 128), jnp.float32),
        mesh=mesh,
        compiler_params=pltpu.CompilerParams(use_tc_tiling_on_sc=False,
                                             needs_layout_passes=False),
        scratch_types=[
            pltpu.VMEM((_CH,), jnp.int32),
            pltpu.VMEM((_CH,), jnp.int32),
            pltpu.VMEM((_CH, _EMB), jnp.float32),
            pltpu.VMEM((_CH, _EMB), jnp.float32),
            pltpu.VMEM((_EMB, _PITCH), jnp.float32),
            pltpu.SemaphoreType.DMA,
            pltpu.SemaphoreType.DMA,
        ],
    )
    def sc_kernel(table_hbm, idx_hbm, out_hbm,
                  idx0, idx1, rows0, rows1, outT_v, sem0, sem1):
        wid = lax.axis_index("s") * _NC + lax.axis_index("c")
        g0 = wid * _BPW
        iota16 = lax.iota(jnp.int32, 16)
        row_ids = [iota16 + h0 * 16 for h0 in range(_EMB // 16)]

        def issue(g, idx_v, rows_v, sem):
            l = g // _BG
            bg = g % _BG
            pltpu.sync_copy(idx_hbm.at[l, pl.ds(bg * _CH, _CH)], idx_v)
            pltpu.async_copy(table_hbm.at[idx_v], rows_v, sem)

        def finish(g, idx_v, rows_v, sem):
            l = g // _BG
            bg = g % _BG
            pltpu.make_async_copy(table_hbm.at[idx_v], rows_v, sem).wait()

            @plsc.parallel_loop(0, _CH, unroll=16)
            def _(r):
                col = jnp.full((16,), r, jnp.int32)
                for h0 in range(_EMB // 16):
                    x = rows_v[r, pl.ds(h0 * 16, 16)]
                    plsc.store_scatter(outT_v, [row_ids[h0], col], x)

            for btp in range(4):
                pltpu.sync_copy(
                    outT_v.at[:, pl.ds(btp * 128, 128)],
                    out_hbm.at[:, l // 8, bg * 4 + btp, l % 8, :],
                )

        issue(g0 + 0, idx0, rows0, sem0)
        issue(g0 + 1, idx1, rows1, sem1)

        @pl.loop(0, _BPW // 2 - 1)
        def _(i):
            e = g0 + 2 * i
            finish(e, idx0, rows0, sem0)
            issue(e + 2, idx0, rows0, sem0)
            finish(e + 1, idx1, rows1, sem1)
            issue(e + 3, idx1, rows1, sem1)

        finish(g0 + _BPW - 2, idx0, rows0, sem0)
        finish(g0 + _BPW - 1, idx1, rows1, sem1)

    out5 = sc_kernel(table, idx_t)
    return out5.transpose(2, 4, 0, 1, 3).reshape(_B, _EMB, _L)
